# baseline (speedup 1.0000x reference)
import functools

import jax
import jax.numpy as jnp
import numpy as np
from jax import lax
from jax.experimental import pallas as pl
from jax.experimental.pallas import tpu as pltpu
from jax.experimental.pallas import tpu_sc as plsc

N = 128 * 32768
NW = 32
LANES = 16
CHUNK = N // NW
SUB = 16384
NSUB = CHUNK // SUB
VPS = SUB // LANES
PROWS = 64

_MININT = np.int32(-2147483648)


def _mesh():
    return plsc.VectorSubcoreMesh(core_axis_name="c", subcore_axis_name="s")


def _monotone_key(x):
    t = jnp.bitwise_and(jnp.right_shift(x, 31), np.int32(0x7FFFFFFF))
    return jnp.bitwise_xor(x, t)


def _ud_key(x):
    return jnp.bitwise_xor(
        x, jnp.bitwise_or(jnp.right_shift(x, 31), _MININT))


UNROLL = 8


def _make_hist(sel_shift, dig_shift, nbuckets, has_sel):
    @functools.partial(
        pl.kernel,
        mesh=_mesh(),
        out_type=jax.ShapeDtypeStruct((NW, nbuckets), jnp.int32),
        scratch_types=[
            pltpu.VMEM((SUB,), jnp.int32),
            pltpu.VMEM((SUB,), jnp.int32),
            pltpu.VMEM((nbuckets,), jnp.int32),
            pltpu.VMEM((PROWS, LANES), jnp.int32),
            pltpu.SemaphoreType.DMA,
            pltpu.SemaphoreType.DMA,
        ],
        compiler_params=pltpu.CompilerParams(needs_layout_passes=False),
    )
    def hist_kernel(mi_hbm, params_hbm, h_hbm, buf0, buf1, hist, pbuf,
                    sem0, sem1):
        wid = lax.axis_index("s") * 2 + lax.axis_index("c")
        zeros16 = jnp.zeros((LANES,), jnp.int32)

        @plsc.parallel_loop(0, nbuckets, step=LANES, unroll=UNROLL)
        def _(d):
            hist[pl.ds(d, LANES)] = zeros16

        if has_sel:
            pltpu.sync_copy(params_hbm, pbuf)
            selv = pbuf[32, :]

        ones16 = jnp.ones((LANES,), jnp.int32)
        dmask = np.int32(nbuckets - 1)
        dshift = jnp.full((LANES,), dig_shift, jnp.int32)
        if has_sel:
            sshift = jnp.full((LANES,), sel_shift, jnp.int32)

        bufs = (buf0, buf1)
        sems = (sem0, sem1)
        base0 = wid * CHUNK
        pending = pltpu.async_copy(mi_hbm.at[pl.ds(base0, SUB)],
                                   bufs[0], sems[0])
        for j in range(NSUB):
            buf = bufs[j % 2]
            pending.wait()
            if j + 1 < NSUB:
                pending = pltpu.async_copy(
                    mi_hbm.at[pl.ds(base0 + (j + 1) * SUB, SUB)],
                    bufs[(j + 1) % 2], sems[(j + 1) % 2])

            @plsc.parallel_loop(0, VPS, unroll=UNROLL)
            def _(v, buf=buf):
                x = buf[pl.ds(v * LANES, LANES)]
                ud = _ud_key(x)
                if dig_shift > 0:
                    slot = lax.shift_right_logical(ud, dshift)
                else:
                    slot = jnp.bitwise_and(ud, dmask)
                if has_sel:
                    sel = lax.shift_right_logical(ud, sshift) == selv
                    plsc.addupdate_scatter(hist, [slot], ones16, mask=sel)
                else:
                    plsc.addupdate_scatter(hist, [slot], ones16)

        pltpu.sync_copy(hist, h_hbm.at[wid])

    return hist_kernel


def _make_merge(level, nbuckets, grid_g, grid_w):
    M = nbuckets
    f32 = np.float32
    hi = jax.lax.Precision.HIGHEST

    def body(h_ref, pin_ref, out_ref):
        hf = h_ref[...].astype(f32)
        onesr = jnp.ones((1, NW), f32)
        totr = jnp.dot(onesr, hf, precision=hi)
        s = jnp.concatenate(
            [totr[:, g * grid_w:(g + 1) * grid_w] for g in range(grid_g)],
            axis=0)
        onc = jnp.ones((grid_w, 1), f32)
        rowsum = jnp.dot(s, onc, precision=hi)
        rg = lax.broadcasted_iota(jnp.int32, (grid_g, grid_g), 0)
        cg = lax.broadcasted_iota(jnp.int32, (grid_g, grid_g), 1)
        lg = jnp.where(cg < rg, f32(1), f32(0))
        a = jnp.dot(lg, rowsum, precision=hi)
        rw = lax.broadcasted_iota(jnp.int32, (grid_w, grid_w), 0)
        cw = lax.broadcasted_iota(jnp.int32, (grid_w, grid_w), 1)
        mw = jnp.where(rw < cw, f32(1), f32(0))
        bexc = jnp.dot(s, mw, precision=hi)
        clx = a + bexc
        cin = clx + s

        if level == 1:
            p = pin_ref[0, 0]
            kf = jnp.floor((np.float32(1.0) - p) * np.float32(N))
            k = kf.astype(jnp.int32)
            prefix = np.int32(0)
            rrem = jnp.minimum(k, np.int32(N - 1))
        else:
            k = pin_ref[33, 0]
            prefix = pin_ref[32, 0]
            rrem = pin_ref[34, 0]
        rf = rrem.astype(f32)

        leq = cin <= rf
        b = jnp.sum(jnp.where(leq, 1, 0).astype(jnp.int32))
        cl = jnp.sum(jnp.where(leq, s, f32(0))).astype(jnp.int32)
        rrem2 = rrem - cl

        if level == 1:
            prefix2 = b
            qrows = jnp.zeros((NW, LANES), jnp.int32)
        else:
            t_ud = jnp.bitwise_or(jnp.left_shift(prefix, 16), b)
            prefix2 = jnp.bitwise_xor(t_ud, _MININT)
            posd = lax.broadcasted_iota(jnp.int32, (M, 1), 0)
            indc = jnp.where(posd == b, f32(1), f32(0))
            eqc = jnp.dot(hf, indc, precision=hi)
            rs = lax.broadcasted_iota(jnp.int32, (NW, NW), 0)
            cs = lax.broadcasted_iota(jnp.int32, (NW, NW), 1)
            ls = jnp.where(cs < rs, f32(1), f32(0))
            exc = jnp.dot(ls, eqc, precision=hi)
            rstar = jnp.minimum(k, np.int32(N - 1))
            need = (k - (rstar - rrem2)).astype(f32)
            qf = jnp.clip(need - exc, f32(0), eqc)
            qrows = jnp.broadcast_to(qf.astype(jnp.int32), (NW, LANES))

        pref_row = jnp.full((1, LANES), prefix2, jnp.int32)
        k_row = jnp.full((1, LANES), k, jnp.int32)
        rrem_row = jnp.full((1, LANES), rrem2, jnp.int32)
        pad = jnp.zeros((PROWS - NW - 3, LANES), jnp.int32)
        out_ref[...] = jnp.concatenate(
            [qrows, pref_row, k_row, rrem_row, pad], axis=0)

    pin_shape = ((1, 1) if level == 1 else (PROWS, LANES))
    pin_dtype = (jnp.float32 if level == 1 else jnp.int32)

    def run(h, pin):
        h2d = h.reshape(NW, M)
        return pl.pallas_call(
            body,
            out_shape=jax.ShapeDtypeStruct((PROWS, LANES), jnp.int32),
        )(h2d, pin.reshape(pin_shape).astype(pin_dtype))

    return run


@functools.partial(
    pl.kernel,
    mesh=_mesh(),
    out_type=jax.ShapeDtypeStruct((N,), jnp.float32),
    scratch_types=[
        pltpu.VMEM((SUB,), jnp.int32),
        pltpu.VMEM((SUB,), jnp.int32),
        pltpu.VMEM((SUB,), jnp.float32),
        pltpu.VMEM((SUB,), jnp.float32),
        pltpu.VMEM((SUB,), jnp.float32),
        pltpu.VMEM((SUB,), jnp.float32),
        pltpu.VMEM((PROWS, LANES), jnp.int32),
        pltpu.SemaphoreType.DMA,
        pltpu.SemaphoreType.DMA,
        pltpu.SemaphoreType.DMA,
        pltpu.SemaphoreType.DMA,
        pltpu.SemaphoreType.DMA,
        pltpu.SemaphoreType.DMA,
    ],
    compiler_params=pltpu.CompilerParams(needs_layout_passes=False),
)
def _apply_kernel(mi_hbm, w_hbm, params_hbm, out_hbm, mbuf0, mbuf1,
                  wbuf0, wbuf1, obuf0, obuf1, pbuf,
                  msem0, msem1, wsem0, wsem1, osem0, osem1):
    wid = lax.axis_index("s") * 2 + lax.axis_index("c")
    pltpu.sync_copy(params_hbm, pbuf)
    tv = pbuf[32, :]
    qv = pbuf[wid, :]
    zf = jnp.zeros((LANES,), jnp.float32)

    mbufs, wbufs, obufs = (mbuf0, mbuf1), (wbuf0, wbuf1), (obuf0, obuf1)
    msems, wsems, osems = (msem0, msem1), (wsem0, wsem1), (osem0, osem1)
    base0 = wid * CHUNK
    pend_m = pltpu.async_copy(mi_hbm.at[pl.ds(base0, SUB)],
                              mbufs[0], msems[0])
    pend_w = pltpu.async_copy(w_hbm.at[pl.ds(base0, SUB)],
                              wbufs[0], wsems[0])
    pend_o = [None, None]
    rv = jnp.zeros((LANES,), jnp.int32)
    for j in range(NSUB):
        mbuf, wbuf, obuf = mbufs[j % 2], wbufs[j % 2], obufs[j % 2]
        pend_m.wait()
        pend_w.wait()
        if j + 1 < NSUB:
            nbase = base0 + (j + 1) * SUB
            pend_m = pltpu.async_copy(mi_hbm.at[pl.ds(nbase, SUB)],
                                      mbufs[(j + 1) % 2], msems[(j + 1) % 2])
            pend_w = pltpu.async_copy(w_hbm.at[pl.ds(nbase, SUB)],
                                      wbufs[(j + 1) % 2], wsems[(j + 1) % 2])
        if pend_o[j % 2] is not None:
            pend_o[j % 2].wait()

        @plsc.parallel_loop(0, VPS, unroll=2 * UNROLL, carry=rv)
        def rv(v, rv, mbuf=mbuf, wbuf=wbuf, obuf=obuf):
            o = v * LANES
            x = mbuf[pl.ds(o, LANES)]
            key = _monotone_key(x)
            less = key < tv
            eq = key == tv
            eqi = jnp.where(eq, np.int32(1), np.int32(0))
            excl = plsc.cumsum(eqi) - eqi
            pc = plsc.all_reduce_population_count(eq)
            zero = jnp.logical_or(less,
                                  jnp.logical_and(eq, (rv + excl) < qv))
            wv = wbuf[pl.ds(o, LANES)]
            obuf[pl.ds(o, LANES)] = jnp.where(zero, zf, wv)
            return rv + pc

        pend_o[j % 2] = pltpu.async_copy(
            obuf, out_hbm.at[pl.ds(base0 + j * SUB, SUB)], osems[j % 2])
    for d in pend_o:
        if d is not None:
            d.wait()


_hist1 = _make_hist(None, 16, 65536, False)
_hist2 = _make_hist(16, 0, 65536, True)
_merge1 = _make_merge(1, 65536, 64, 1024)
_merge2 = _make_merge(2, 65536, 64, 1024)


def kernel(weight, mask_weights, masking_percent):
    mi = lax.bitcast_convert_type(mask_weights.reshape(-1), jnp.int32)
    wf = weight.reshape(-1)
    dummy = jnp.zeros((PROWS, LANES), jnp.int32)

    h1 = _hist1(mi, dummy)
    p1 = _merge1(h1, masking_percent)
    h2 = _hist2(mi, p1)
    p2 = _merge2(h2, p1)
    out = _apply_kernel(mi, wf, p2)
    return out.reshape(weight.shape)


# speedup vs baseline: 1.2840x; 1.2840x over previous
import functools

import jax
import jax.numpy as jnp
import numpy as np
from jax import lax
from jax.experimental import pallas as pl
from jax.experimental.pallas import tpu as pltpu
from jax.experimental.pallas import tpu_sc as plsc

N = 128 * 32768
NW = 32
LANES = 16
CHUNK = N // NW
SUB = 16384
NSUB = CHUNK // SUB
VPS = SUB // LANES
PROWS = 64

_MININT = np.int32(-2147483648)


def _mesh():
    return plsc.VectorSubcoreMesh(core_axis_name="c", subcore_axis_name="s")


def _monotone_key(x):
    t = jnp.bitwise_and(jnp.right_shift(x, 31), np.int32(0x7FFFFFFF))
    return jnp.bitwise_xor(x, t)


def _ud_key(x):
    return jnp.bitwise_xor(
        x, jnp.bitwise_or(jnp.right_shift(x, 31), _MININT))


UNROLL = 8


def _make_hist(sel_shift, dig_shift, nbuckets, has_sel):
    @functools.partial(
        pl.kernel,
        mesh=_mesh(),
        out_type=jax.ShapeDtypeStruct((NW, nbuckets), jnp.int32),
        scratch_types=[
            pltpu.VMEM((SUB,), jnp.int32),
            pltpu.VMEM((SUB,), jnp.int32),
            pltpu.VMEM((nbuckets,), jnp.int32),
            pltpu.VMEM((PROWS, LANES), jnp.int32),
            pltpu.SemaphoreType.DMA,
            pltpu.SemaphoreType.DMA,
        ],
        compiler_params=pltpu.CompilerParams(needs_layout_passes=False),
    )
    def hist_kernel(mi_hbm, params_hbm, h_hbm, buf0, buf1, hist, pbuf,
                    sem0, sem1):
        wid = lax.axis_index("s") * 2 + lax.axis_index("c")
        zeros16 = jnp.zeros((LANES,), jnp.int32)

        @plsc.parallel_loop(0, nbuckets, step=LANES, unroll=UNROLL)
        def _(d):
            hist[pl.ds(d, LANES)] = zeros16

        if has_sel:
            pltpu.sync_copy(params_hbm, pbuf)
            selv = pbuf[32, :]

        ones16 = jnp.ones((LANES,), jnp.int32)
        dmask = np.int32(nbuckets - 1)
        dshift = jnp.full((LANES,), dig_shift, jnp.int32)
        if has_sel:
            sshift = jnp.full((LANES,), sel_shift, jnp.int32)

        bufs = (buf0, buf1)
        sems = (sem0, sem1)
        base0 = wid * CHUNK
        pending = pltpu.async_copy(mi_hbm.at[pl.ds(base0, SUB)],
                                   bufs[0], sems[0])
        for j in range(NSUB):
            buf = bufs[j % 2]
            pending.wait()
            if j + 1 < NSUB:
                pending = pltpu.async_copy(
                    mi_hbm.at[pl.ds(base0 + (j + 1) * SUB, SUB)],
                    bufs[(j + 1) % 2], sems[(j + 1) % 2])

            @plsc.parallel_loop(0, VPS, unroll=UNROLL)
            def _(v, buf=buf):
                x = buf[pl.ds(v * LANES, LANES)]
                ud = _ud_key(x)
                if dig_shift > 0:
                    slot = lax.shift_right_logical(ud, dshift)
                else:
                    slot = jnp.bitwise_and(ud, dmask)
                if has_sel:
                    sel = lax.shift_right_logical(ud, sshift) == selv
                    plsc.addupdate_scatter(hist, [slot], ones16, mask=sel)
                else:
                    plsc.addupdate_scatter(hist, [slot], ones16)

        pltpu.sync_copy(hist, h_hbm.at[wid])

    return hist_kernel


def _make_merge(level, nbuckets, grid_g, grid_w):
    M = nbuckets
    f32 = np.float32
    hi = jax.lax.Precision.HIGHEST

    def body(h_ref, pin_ref, out_ref):
        hf = h_ref[...].astype(f32)
        onesr = jnp.ones((1, NW), f32)
        totr = jnp.dot(onesr, hf, precision=hi)
        s = jnp.concatenate(
            [totr[:, g * grid_w:(g + 1) * grid_w] for g in range(grid_g)],
            axis=0)
        onc = jnp.ones((grid_w, 1), f32)
        rowsum = jnp.dot(s, onc, precision=hi)
        rg = lax.broadcasted_iota(jnp.int32, (grid_g, grid_g), 0)
        cg = lax.broadcasted_iota(jnp.int32, (grid_g, grid_g), 1)
        lg = jnp.where(cg < rg, f32(1), f32(0))
        a = jnp.dot(lg, rowsum, precision=hi)
        rw = lax.broadcasted_iota(jnp.int32, (grid_w, grid_w), 0)
        cw = lax.broadcasted_iota(jnp.int32, (grid_w, grid_w), 1)
        mw = jnp.where(rw < cw, f32(1), f32(0))
        bexc = jnp.dot(s, mw, precision=hi)
        clx = a + bexc
        cin = clx + s

        if level == 1:
            p = pin_ref[0, 0]
            kf = jnp.floor((np.float32(1.0) - p) * np.float32(N))
            k = kf.astype(jnp.int32)
            prefix = np.int32(0)
            rrem = jnp.minimum(k, np.int32(N - 1))
        else:
            k = pin_ref[33, 0]
            prefix = pin_ref[32, 0]
            rrem = pin_ref[34, 0]
        rf = rrem.astype(f32)

        leq = cin <= rf
        b = jnp.sum(jnp.where(leq, 1, 0).astype(jnp.int32))
        cl = jnp.sum(jnp.where(leq, s, f32(0))).astype(jnp.int32)
        rrem2 = rrem - cl

        if level == 1:
            prefix2 = b
            qrows = jnp.zeros((NW, LANES), jnp.int32)
        else:
            t_ud = jnp.bitwise_or(jnp.left_shift(prefix, 16), b)
            prefix2 = jnp.bitwise_xor(t_ud, _MININT)
            posd = lax.broadcasted_iota(jnp.int32, (M, 1), 0)
            indc = jnp.where(posd == b, f32(1), f32(0))
            eqc = jnp.dot(hf, indc, precision=hi)
            rs = lax.broadcasted_iota(jnp.int32, (NW, NW), 0)
            cs = lax.broadcasted_iota(jnp.int32, (NW, NW), 1)
            ls = jnp.where(cs < rs, f32(1), f32(0))
            exc = jnp.dot(ls, eqc, precision=hi)
            rstar = jnp.minimum(k, np.int32(N - 1))
            need = (k - (rstar - rrem2)).astype(f32)
            qf = jnp.clip(need - exc, f32(0), eqc)
            qrows = jnp.broadcast_to(qf.astype(jnp.int32), (NW, LANES))

        pref_row = jnp.full((1, LANES), prefix2, jnp.int32)
        k_row = jnp.full((1, LANES), k, jnp.int32)
        rrem_row = jnp.full((1, LANES), rrem2, jnp.int32)
        pad = jnp.zeros((PROWS - NW - 3, LANES), jnp.int32)
        out_ref[...] = jnp.concatenate(
            [qrows, pref_row, k_row, rrem_row, pad], axis=0)

    pin_shape = ((1, 1) if level == 1 else (PROWS, LANES))
    pin_dtype = (jnp.float32 if level == 1 else jnp.int32)

    def run(h, pin):
        h2d = h.reshape(NW, M)
        return pl.pallas_call(
            body,
            out_shape=jax.ShapeDtypeStruct((PROWS, LANES), jnp.int32),
        )(h2d, pin.reshape(pin_shape).astype(pin_dtype))

    return run


@functools.partial(
    pl.kernel,
    mesh=_mesh(),
    out_type=jax.ShapeDtypeStruct((N,), jnp.float32),
    scratch_types=[
        pltpu.VMEM((SUB,), jnp.int32),
        pltpu.VMEM((SUB,), jnp.int32),
        pltpu.VMEM((SUB,), jnp.float32),
        pltpu.VMEM((SUB,), jnp.float32),
        pltpu.VMEM((SUB,), jnp.float32),
        pltpu.VMEM((SUB,), jnp.float32),
        pltpu.VMEM((PROWS, LANES), jnp.int32),
        pltpu.SemaphoreType.DMA,
        pltpu.SemaphoreType.DMA,
        pltpu.SemaphoreType.DMA,
        pltpu.SemaphoreType.DMA,
        pltpu.SemaphoreType.DMA,
        pltpu.SemaphoreType.DMA,
    ],
    compiler_params=pltpu.CompilerParams(needs_layout_passes=False),
)
def _apply_kernel(mi_hbm, w_hbm, params_hbm, out_hbm, mbuf0, mbuf1,
                  wbuf0, wbuf1, obuf0, obuf1, pbuf,
                  msem0, msem1, wsem0, wsem1, osem0, osem1):
    wid = lax.axis_index("s") * 2 + lax.axis_index("c")
    pltpu.sync_copy(params_hbm, pbuf)
    tv = pbuf[32, :]
    qv = pbuf[wid, :]
    zf = jnp.zeros((LANES,), jnp.float32)

    mbufs, wbufs, obufs = (mbuf0, mbuf1), (wbuf0, wbuf1), (obuf0, obuf1)
    msems, wsems, osems = (msem0, msem1), (wsem0, wsem1), (osem0, osem1)
    base0 = wid * CHUNK
    pend_m = pltpu.async_copy(mi_hbm.at[pl.ds(base0, SUB)],
                              mbufs[0], msems[0])
    pend_w = pltpu.async_copy(w_hbm.at[pl.ds(base0, SUB)],
                              wbufs[0], wsems[0])
    pend_o = [None, None]
    rv = jnp.zeros((LANES,), jnp.int32)
    for j in range(NSUB):
        mbuf, wbuf, obuf = mbufs[j % 2], wbufs[j % 2], obufs[j % 2]
        pend_m.wait()
        pend_w.wait()
        if j + 1 < NSUB:
            nbase = base0 + (j + 1) * SUB
            pend_m = pltpu.async_copy(mi_hbm.at[pl.ds(nbase, SUB)],
                                      mbufs[(j + 1) % 2], msems[(j + 1) % 2])
            pend_w = pltpu.async_copy(w_hbm.at[pl.ds(nbase, SUB)],
                                      wbufs[(j + 1) % 2], wsems[(j + 1) % 2])
        if pend_o[j % 2] is not None:
            pend_o[j % 2].wait()

        @plsc.parallel_loop(0, VPS, unroll=UNROLL // 2, carry=rv)
        def rv(v, rv, mbuf=mbuf, wbuf=wbuf, obuf=obuf):
            o = v * LANES
            x = mbuf[pl.ds(o, LANES)]
            key = _monotone_key(x)
            less = key < tv
            eq = key == tv
            eqi = jnp.where(eq, np.int32(1), np.int32(0))
            excl = plsc.cumsum(eqi) - eqi
            pc = plsc.all_reduce_population_count(eq)
            zero = jnp.logical_or(less,
                                  jnp.logical_and(eq, (rv + excl) < qv))
            wv = wbuf[pl.ds(o, LANES)]
            obuf[pl.ds(o, LANES)] = jnp.where(zero, zf, wv)
            return rv + pc

        pend_o[j % 2] = pltpu.async_copy(
            obuf, out_hbm.at[pl.ds(base0 + j * SUB, SUB)], osems[j % 2])
    for d in pend_o:
        if d is not None:
            d.wait()


_hist1 = _make_hist(None, 16, 65536, False)
_hist2 = _make_hist(16, 0, 65536, True)
_merge1 = _make_merge(1, 65536, 64, 1024)
_merge2 = _make_merge(2, 65536, 64, 1024)


def kernel(weight, mask_weights, masking_percent):
    mi = lax.bitcast_convert_type(mask_weights.reshape(-1), jnp.int32)
    wf = weight.reshape(-1)
    dummy = jnp.zeros((PROWS, LANES), jnp.int32)

    h1 = _hist1(mi, dummy)
    p1 = _merge1(h1, masking_percent)
    h2 = _hist2(mi, p1)
    p2 = _merge2(h2, p1)
    out = _apply_kernel(mi, wf, p2)
    return out.reshape(weight.shape)
